# Initial kernel scaffold; baseline (speedup 1.0000x reference)
import jax
import jax.numpy as jnp
from jax.experimental import pallas as pl


def kernel(x, edge_index, W1, b1, W2, b2, eps):
    raise NotImplementedError("write your pallas kernel here")


# speedup vs baseline: 7.6707x; 7.6707x over previous
import functools

import jax
import jax.numpy as jnp
from jax import lax
from jax.experimental import pallas as pl
from jax.experimental.pallas import tpu as pltpu
from jax.experimental.pallas import tpu_sc as plsc

N = 10000
E = 320000
D = 128

NC = 2
NS = 16
NW = NC * NS
EPW = E // NW
CH = 80
NCHUNK = EPW // CH
NP = 10240
RPT = NP // NS


def _sc_partial_segment_sum(x, src3, dst3, zeros_slab):
    mesh = plsc.VectorSubcoreMesh(
        core_axis_name="c", subcore_axis_name="s", num_cores=NC,
        num_subcores=NS)

    @functools.partial(
        pl.kernel,
        out_type=jax.ShapeDtypeStruct((NC, NP, D), jnp.float32),
        mesh=mesh,
        scratch_types=dict(
            sidx=pltpu.VMEM((NCHUNK, CH), jnp.int32),
            didx=pltpu.VMEM((NCHUNK, CH), jnp.int32),
            rows=pltpu.VMEM((CH, D), jnp.float32),
            acc=pltpu.VMEM_SHARED((NP, D), jnp.float32),
            sem=pltpu.SemaphoreType.DMA,
        ),
    )
    def sc_kernel(x_hbm, src_hbm, dst_hbm, zero_hbm, out_hbm,
                  sidx, didx, rows, acc, sem):
        c = lax.axis_index("c")
        s = lax.axis_index("s")
        wid = c * NS + s

        pltpu.sync_copy(src_hbm.at[wid], sidx)
        pltpu.sync_copy(dst_hbm.at[wid], didx)
        pltpu.sync_copy(zero_hbm, acc.at[pl.ds(s * RPT, RPT)])
        plsc.subcore_barrier()

        def body(j, carry):
            pltpu.async_copy(x_hbm.at[sidx.at[j]], rows, sem).wait()
            pltpu.sync_copy(rows, acc.at[didx.at[j]], add=True)
            return carry

        lax.fori_loop(0, NCHUNK, body, 0)
        plsc.subcore_barrier()
        pltpu.sync_copy(acc.at[pl.ds(s * RPT, RPT)],
                        out_hbm.at[c].at[pl.ds(s * RPT, RPT)])

    return sc_kernel(x, src3, dst3, zeros_slab)


BN = 2000


def _tc_mlp_body(p_ref, x_ref, w1_ref, b1_ref, w2_ref, b2_ref, e_ref,
                 o_ref):
    scale = 1.0 + e_ref[0, 0]
    y = p_ref[0] + p_ref[1] + scale * x_ref[...]
    h = jnp.maximum(
        jnp.dot(y, w1_ref[...], preferred_element_type=jnp.float32)
        + b1_ref[...], 0.0)
    o_ref[...] = (
        jnp.dot(h, w2_ref[...], preferred_element_type=jnp.float32)
        + b2_ref[...])


def _tc_mlp(partial, x, W1, b1, W2, b2, eps):
    grid = (N // BN,)
    return pl.pallas_call(
        _tc_mlp_body,
        grid=grid,
        in_specs=[
            pl.BlockSpec((NC, BN, D), lambda i: (0, i, 0)),
            pl.BlockSpec((BN, D), lambda i: (i, 0)),
            pl.BlockSpec((D, D), lambda i: (0, 0)),
            pl.BlockSpec((1, D), lambda i: (0, 0)),
            pl.BlockSpec((D, D), lambda i: (0, 0)),
            pl.BlockSpec((1, D), lambda i: (0, 0)),
            pl.BlockSpec((1, 1), lambda i: (0, 0)),
        ],
        out_specs=pl.BlockSpec((BN, D), lambda i: (i, 0)),
        out_shape=jax.ShapeDtypeStruct((N, D), jnp.float32),
    )(partial, x, W1, b1.reshape(1, D), W2, b2.reshape(1, D),
      eps.reshape(1, 1))


def kernel(x, edge_index, W1, b1, W2, b2, eps):
    src3 = edge_index[0].reshape(NW, NCHUNK, CH)
    dst3 = edge_index[1].reshape(NW, NCHUNK, CH)
    zeros_slab = jnp.zeros((RPT, D), jnp.float32)
    partial = _sc_partial_segment_sum(x, src3, dst3, zeros_slab)
    return _tc_mlp(partial, x, W1, b1, W2, b2, eps)


# speedup vs baseline: 11.6435x; 1.5179x over previous
import functools

import jax
import jax.numpy as jnp
from jax import lax
from jax.experimental import pallas as pl
from jax.experimental.pallas import tpu as pltpu
from jax.experimental.pallas import tpu_sc as plsc

N = 10000
E = 320000
D = 128

NC = 2
NS = 16
NW = NC * NS
EPW = E // NW
CH = 40
NCHUNK = EPW // CH
NBUF = 5
NROUND = NCHUNK // NBUF
NP = 10240
RPT = NP // NS


def _sc_partial_segment_sum(x, src3, dst3, zeros_slab):
    mesh = plsc.VectorSubcoreMesh(
        core_axis_name="c", subcore_axis_name="s", num_cores=NC,
        num_subcores=NS)

    @functools.partial(
        pl.kernel,
        out_type=jax.ShapeDtypeStruct((NC, NP, D), jnp.float32),
        mesh=mesh,
        scratch_types=dict(
            sbuf=[pltpu.VMEM((NBUF, CH), jnp.int32) for _ in range(2)],
            dbuf=[pltpu.VMEM((NBUF, CH), jnp.int32) for _ in range(2)],
            rows=[pltpu.VMEM((CH, D), jnp.float32) for _ in range(NBUF)],
            acc=pltpu.VMEM_SHARED((NP, D), jnp.float32),
            gsem=[pltpu.SemaphoreType.DMA for _ in range(NBUF)],
            ssem=[pltpu.SemaphoreType.DMA for _ in range(NBUF)],
            isem=[pltpu.SemaphoreType.DMA for _ in range(2)],
        ),
    )
    def sc_kernel(x_hbm, src_hbm, dst_hbm, zero_hbm, out_hbm,
                  sbuf, dbuf, rows, acc, gsem, ssem, isem):
        c = lax.axis_index("c")
        s = lax.axis_index("s")
        wid = c * NS + s

        pltpu.sync_copy(zero_hbm, acc.at[pl.ds(s * RPT, RPT)])
        plsc.subcore_barrier()

        pltpu.sync_copy(src_hbm.at[wid].at[0], sbuf[0])
        pltpu.sync_copy(dst_hbm.at[wid].at[0], dbuf[0])
        pltpu.async_copy(src_hbm.at[wid].at[1], sbuf[1], isem[1])
        pltpu.async_copy(dst_hbm.at[wid].at[1], dbuf[1], isem[1])
        for b in range(NBUF):
            pltpu.async_copy(x_hbm.at[sbuf[0].at[b]], rows[b], gsem[b])

        def body(g, carry):
            for par in range(2):
                r = 2 * g + par
                sb, db = sbuf[par], dbuf[par]
                so, do = sbuf[1 - par], dbuf[1 - par]
                scat = []
                for b in range(NBUF):
                    pltpu.make_async_copy(
                        x_hbm.at[sb.at[b]], rows[b], gsem[b]).wait()
                    scat.append(pltpu.async_copy(
                        rows[b], acc.at[db.at[b]], ssem[b], add=True))
                @pl.when(r < NROUND - 1)
                def _():
                    pltpu.make_async_copy(
                        src_hbm.at[wid].at[0], so, isem[1 - par]).wait()
                    pltpu.make_async_copy(
                        dst_hbm.at[wid].at[0], do, isem[1 - par]).wait()
                for b in range(NBUF):
                    scat[b].wait()

                    @pl.when(r < NROUND - 1)
                    def _():
                        pltpu.async_copy(
                            x_hbm.at[so.at[b]], rows[b], gsem[b])
                @pl.when(r < NROUND - 2)
                def _():
                    pltpu.async_copy(
                        src_hbm.at[wid].at[r + 2], sb, isem[par])
                    pltpu.async_copy(
                        dst_hbm.at[wid].at[r + 2], db, isem[par])
            return carry

        lax.fori_loop(0, NROUND // 2, body, 0)
        plsc.subcore_barrier()
        pltpu.sync_copy(acc.at[pl.ds(s * RPT, RPT)],
                        out_hbm.at[c].at[pl.ds(s * RPT, RPT)])

    return sc_kernel(x, src3, dst3, zeros_slab)


BN = 2000


def _tc_mlp_body(p_ref, x_ref, w1_ref, b1_ref, w2_ref, b2_ref, e_ref,
                 o_ref):
    scale = 1.0 + e_ref[0, 0]
    y = p_ref[0] + p_ref[1] + scale * x_ref[...]
    h = jnp.maximum(
        jnp.dot(y, w1_ref[...], preferred_element_type=jnp.float32)
        + b1_ref[...], 0.0)
    o_ref[...] = (
        jnp.dot(h, w2_ref[...], preferred_element_type=jnp.float32)
        + b2_ref[...])


def _tc_mlp(partial, x, W1, b1, W2, b2, eps):
    grid = (N // BN,)
    return pl.pallas_call(
        _tc_mlp_body,
        grid=grid,
        in_specs=[
            pl.BlockSpec((NC, BN, D), lambda i: (0, i, 0)),
            pl.BlockSpec((BN, D), lambda i: (i, 0)),
            pl.BlockSpec((D, D), lambda i: (0, 0)),
            pl.BlockSpec((1, D), lambda i: (0, 0)),
            pl.BlockSpec((D, D), lambda i: (0, 0)),
            pl.BlockSpec((1, D), lambda i: (0, 0)),
            pl.BlockSpec((1, 1), lambda i: (0, 0)),
        ],
        out_specs=pl.BlockSpec((BN, D), lambda i: (i, 0)),
        out_shape=jax.ShapeDtypeStruct((N, D), jnp.float32),
    )(partial, x, W1, b1.reshape(1, D), W2, b2.reshape(1, D),
      eps.reshape(1, 1))


def kernel(x, edge_index, W1, b1, W2, b2, eps):
    src3 = edge_index[0].reshape(NW, NROUND, NBUF, CH)
    dst3 = edge_index[1].reshape(NW, NROUND, NBUF, CH)
    zeros_slab = jnp.zeros((RPT, D), jnp.float32)
    partial = _sc_partial_segment_sum(x, src3, dst3, zeros_slab)
    return _tc_mlp(partial, x, W1, b1, W2, b2, eps)


# speedup vs baseline: 12.4629x; 1.0704x over previous
import functools

import jax
import jax.numpy as jnp
from jax import lax
from jax.experimental import pallas as pl
from jax.experimental.pallas import tpu as pltpu
from jax.experimental.pallas import tpu_sc as plsc

N = 10000
E = 320000
D = 128

NC = 2
NS = 16
NW = NC * NS
EPW = E // NW
CH = 40
NCHUNK = EPW // CH
NBUF = 5
NROUND = NCHUNK // NBUF
NP = 10240
RPT = NP // NS


def _sc_partial_segment_sum(x, ei5, zeros_slab):
    mesh = plsc.VectorSubcoreMesh(
        core_axis_name="c", subcore_axis_name="s", num_cores=NC,
        num_subcores=NS)

    @functools.partial(
        pl.kernel,
        out_type=jax.ShapeDtypeStruct((NC, NP, D), jnp.float32),
        mesh=mesh,
        scratch_types=dict(
            sbuf=[pltpu.VMEM((NBUF, CH), jnp.int32) for _ in range(2)],
            dbuf=[pltpu.VMEM((NBUF, CH), jnp.int32) for _ in range(2)],
            rows=[pltpu.VMEM((CH, D), jnp.float32) for _ in range(NBUF)],
            acc=pltpu.VMEM_SHARED((NP, D), jnp.float32),
            gsem=[pltpu.SemaphoreType.DMA for _ in range(NBUF)],
            ssem=[pltpu.SemaphoreType.DMA for _ in range(NBUF)],
            isem=[pltpu.SemaphoreType.DMA for _ in range(2)],
        ),
    )
    def sc_kernel(x_hbm, ei_hbm, zero_hbm, out_hbm,
                  sbuf, dbuf, rows, acc, gsem, ssem, isem):
        src_hbm = ei_hbm.at[0]
        dst_hbm = ei_hbm.at[1]
        c = lax.axis_index("c")
        s = lax.axis_index("s")
        wid = c * NS + s

        pltpu.sync_copy(zero_hbm, acc.at[pl.ds(s * RPT, RPT)])
        plsc.subcore_barrier()

        pltpu.sync_copy(src_hbm.at[wid].at[0], sbuf[0])
        pltpu.sync_copy(dst_hbm.at[wid].at[0], dbuf[0])
        pltpu.async_copy(src_hbm.at[wid].at[1], sbuf[1], isem[1])
        pltpu.async_copy(dst_hbm.at[wid].at[1], dbuf[1], isem[1])
        for b in range(NBUF):
            pltpu.async_copy(x_hbm.at[sbuf[0].at[b]], rows[b], gsem[b])

        def body(g, carry):
            for par in range(2):
                r = 2 * g + par
                sb, db = sbuf[par], dbuf[par]
                so, do = sbuf[1 - par], dbuf[1 - par]
                scat = []
                for b in range(NBUF):
                    pltpu.make_async_copy(
                        x_hbm.at[sb.at[b]], rows[b], gsem[b]).wait()
                    scat.append(pltpu.async_copy(
                        rows[b], acc.at[db.at[b]], ssem[b], add=True))
                @pl.when(r < NROUND - 1)
                def _():
                    pltpu.make_async_copy(
                        src_hbm.at[wid].at[0], so, isem[1 - par]).wait()
                    pltpu.make_async_copy(
                        dst_hbm.at[wid].at[0], do, isem[1 - par]).wait()
                for b in range(NBUF):
                    scat[b].wait()

                    @pl.when(r < NROUND - 1)
                    def _():
                        pltpu.async_copy(
                            x_hbm.at[so.at[b]], rows[b], gsem[b])
                @pl.when(r < NROUND - 2)
                def _():
                    pltpu.async_copy(
                        src_hbm.at[wid].at[r + 2], sb, isem[par])
                    pltpu.async_copy(
                        dst_hbm.at[wid].at[r + 2], db, isem[par])
            return carry

        lax.fori_loop(0, NROUND // 2, body, 0)
        plsc.subcore_barrier()
        pltpu.sync_copy(acc.at[pl.ds(s * RPT, RPT)],
                        out_hbm.at[c].at[pl.ds(s * RPT, RPT)])

    return sc_kernel(x, ei5, zeros_slab)


BN = 2000


def _tc_mlp_body(p_ref, x_ref, w1_ref, b1_ref, w2_ref, b2_ref, e_ref,
                 o_ref):
    scale = 1.0 + e_ref[0, 0]
    y = p_ref[0] + p_ref[1] + scale * x_ref[...]
    h = jnp.maximum(
        jnp.dot(y, w1_ref[...], preferred_element_type=jnp.float32)
        + b1_ref[...], 0.0)
    o_ref[...] = (
        jnp.dot(h, w2_ref[...], preferred_element_type=jnp.float32)
        + b2_ref[...])


def _tc_mlp(partial, x, W1, b1, W2, b2, eps):
    grid = (N // BN,)
    return pl.pallas_call(
        _tc_mlp_body,
        grid=grid,
        in_specs=[
            pl.BlockSpec((NC, BN, D), lambda i: (0, i, 0)),
            pl.BlockSpec((BN, D), lambda i: (i, 0)),
            pl.BlockSpec((D, D), lambda i: (0, 0)),
            pl.BlockSpec((1, D), lambda i: (0, 0)),
            pl.BlockSpec((D, D), lambda i: (0, 0)),
            pl.BlockSpec((1, D), lambda i: (0, 0)),
            pl.BlockSpec((1, 1), lambda i: (0, 0)),
        ],
        out_specs=pl.BlockSpec((BN, D), lambda i: (i, 0)),
        out_shape=jax.ShapeDtypeStruct((N, D), jnp.float32),
    )(partial, x, W1, b1.reshape(1, D), W2, b2.reshape(1, D),
      eps.reshape(1, 1))


def kernel(x, edge_index, W1, b1, W2, b2, eps):
    ei5 = edge_index.reshape(2, NW, NROUND, NBUF, CH)
    zeros_slab = jnp.zeros((RPT, D), jnp.float32)
    partial = _sc_partial_segment_sum(x, ei5, zeros_slab)
    return _tc_mlp(partial, x, W1, b1, W2, b2, eps)
